# Initial kernel scaffold; baseline (speedup 1.0000x reference)
import jax
import jax.numpy as jnp
from jax.experimental import pallas as pl


def kernel(x, edge_index, edge_attr, batch, W_node, b_node, g_node, be_node, W_edge, b_edge, g_edge, be_edge, Wn1, bn1, root1, bias1, g1, bb1, Wn2, bn2, root2, bias2, g2, bb2, Wp1, bp1, Wp2, bp2):
    raise NotImplementedError("write your pallas kernel here")


# speedup vs baseline: 6.8075x; 6.8075x over previous
import functools

import jax
import jax.numpy as jnp
from jax import lax
from jax.experimental import pallas as pl
from jax.experimental.pallas import tpu as pltpu
from jax.experimental.pallas import tpu_sc as plsc

_N = 10000
_E = 160000
_DF = 128
_DE = 16
_H = 16
_G = 64
_EPS = 1e-5

_NC = 2
_NS = 16
_NW = _NC * _NS
_EPT = _E // _NW
_CH = 125
_NCHUNK = _EPT // _CH
_NPT = _N // _NS

_E8 = _E // 8


def _node_embed_body(x_ref, w_ref, b_ref, g_ref, be_ref, out_ref):
    z = jax.nn.relu(jnp.dot(x_ref[...], w_ref[...],
                            preferred_element_type=jnp.float32) + b_ref[...])
    m = jnp.mean(z, axis=0, keepdims=True)
    v = jnp.mean(z * z, axis=0, keepdims=True) - m * m
    out_ref[...] = g_ref[...] * (z - m) * jax.lax.rsqrt(v + _EPS) + be_ref[...]


def _edge_embed_body(eaf_ref, w_ref, b_ref, g_ref, be_ref, zf_ref, ec_ref,
                     acc_ref):
    i = pl.program_id(0)
    nb = pl.num_programs(0)
    w16 = w_ref[...]
    wrow = jnp.concatenate([w16] * 8, axis=0)
    wbd_t = jnp.concatenate([w16.T] * 8, axis=0)
    wtile = jnp.concatenate([jnp.concatenate([w16] * 8, axis=1)] * 8, axis=0)
    rg = lax.broadcasted_iota(jnp.int32, (128, 128), 0) // 16
    cg = lax.broadcasted_iota(jnp.int32, (128, 128), 1) // 16
    wbd = jnp.where(rg == cg, wtile, 0.0)
    b128 = jnp.concatenate([b_ref[...]] * 8, axis=0)
    z = jax.nn.relu(jnp.dot(eaf_ref[...], wbd,
                            preferred_element_type=jnp.float32) + b128)
    zf_ref[...] = z

    @pl.when(i == 0)
    def _init():
        acc_ref[...] = jnp.zeros_like(acc_ref)

    s1 = jnp.sum(z, axis=0)
    s2 = jnp.sum(z * z, axis=0)
    acc_ref[...] = acc_ref[...] + jnp.stack([s1, s2], axis=0)

    @pl.when(i == nb - 1)
    def _fin():
        tot = acc_ref[...]
        m16 = jnp.zeros((16,), jnp.float32)
        q16 = jnp.zeros((16,), jnp.float32)
        for gidx in range(8):
            m16 = m16 + tot[0, gidx * 16:(gidx + 1) * 16]
            q16 = q16 + tot[1, gidx * 16:(gidx + 1) * 16]
        m = m16 / _E
        v = q16 / _E - m * m
        a = g_ref[...] * jax.lax.rsqrt(v + _EPS)
        c = be_ref[...] - m * a
        ec_ref[...] = jnp.stack([a, c], axis=0)


def _msg_body(act_tanh, zf_ref, xjf_ref, ec_ref, wn_ref, bn_ref, out_ref):
    a = ec_ref[0, :]
    c = ec_ref[1, :]
    wn_eff = a[:, None] * wn_ref[...]
    bn_eff = bn_ref[...] + jnp.dot(c[None, :], wn_ref[...],
                                   preferred_element_type=jnp.float32)[0]
    ri = lax.broadcasted_iota(jnp.int32, (16, 256), 0)
    rj = lax.broadcasted_iota(jnp.int32, (16, 256), 1)
    rmat = (rj // 16 == ri).astype(jnp.float32)
    si = lax.broadcasted_iota(jnp.int32, (256, 16), 0)
    sj = lax.broadcasted_iota(jnp.int32, (256, 16), 1)
    smat = (si % 16 == sj).astype(jnp.float32)
    slots = []
    for gidx in range(8):
        eg = zf_ref[:, gidx * 16:(gidx + 1) * 16]
        xg = xjf_ref[:, gidx * 16:(gidx + 1) * 16]
        t = jnp.dot(eg, wn_eff, preferred_element_type=jnp.float32) + bn_eff
        t = jnp.tanh(t) if act_tanh else jax.nn.relu(t)
        xr = jnp.dot(xg, rmat, preferred_element_type=jnp.float32)
        slots.append(jnp.dot(t * xr, smat, preferred_element_type=jnp.float32))
    out_ref[...] = jnp.concatenate(slots, axis=1)


def _update_body(h_ref, sp_ref, cp_ref, root_ref, bias_ref, g_ref, bb_ref,
                 out_ref):
    s = sp_ref[0] + sp_ref[1]
    cnt = cp_ref[0, :, 0:1] + cp_ref[1, :, 0:1]
    agg = s / jnp.maximum(cnt, 1.0)
    u = jnp.dot(h_ref[...], root_ref[...],
                preferred_element_type=jnp.float32) + agg + bias_ref[...]
    m = jnp.mean(u, axis=0, keepdims=True)
    v = jnp.mean(u * u, axis=0, keepdims=True) - m * m
    out_ref[...] = g_ref[...] * (u - m) * jax.lax.rsqrt(v + _EPS) + bb_ref[...]


def _final_body(hf_ref, spf_ref, cpf_ref, bf_ref, root_ref, bias_ref, g_ref,
                bb_ref, wp1_ref, bp1_ref, wp2_ref, bp2_ref, out_ref):
    root = root_ref[...]
    rtile = jnp.concatenate([jnp.concatenate([root] * 8, axis=1)] * 8, axis=0)
    rg = lax.broadcasted_iota(jnp.int32, (128, 128), 0) // 16
    cg = lax.broadcasted_iota(jnp.int32, (128, 128), 1) // 16
    rbd = jnp.where(rg == cg, rtile, 0.0)
    sf = spf_ref[0] + spf_ref[1]
    cntf = cpf_ref[0] + cpf_ref[1]
    aggf = sf / jnp.maximum(cntf, 1.0)
    bias128 = jnp.concatenate([bias_ref[...]] * 8, axis=0)
    uf = jnp.dot(hf_ref[...], rbd,
                 preferred_element_type=jnp.float32) + aggf + bias128
    s128 = jnp.sum(uf, axis=0)
    q128 = jnp.sum(uf * uf, axis=0)
    m16 = jnp.zeros((16,), jnp.float32)
    q16 = jnp.zeros((16,), jnp.float32)
    for k in range(8):
        m16 = m16 + s128[k * 16:(k + 1) * 16]
        q16 = q16 + q128[k * 16:(k + 1) * 16]
    m16 = m16 / _N
    v16 = q16 / _N - m16 * m16
    a16 = g_ref[...] * jax.lax.rsqrt(v16 + _EPS)
    c16 = bb_ref[...] - m16 * a16
    a128 = jnp.concatenate([a16] * 8, axis=0)
    c128 = jnp.concatenate([c16] * 8, axis=0)
    h2f = uf * a128 + c128

    bf = bf_ref[...]
    nr = _N // 8
    gids = lax.broadcasted_iota(jnp.int32, (nr, _G), 1)
    ssum = jnp.zeros((_G, _H), jnp.float32)
    cnt_b = jnp.zeros((_G, 1), jnp.float32)
    for k in range(8):
        bk = bf[:, k * 16:k * 16 + 1]
        ohk = (bk == gids).astype(jnp.float32)
        hk = h2f[:, k * 16:(k + 1) * 16]
        ssum = ssum + jax.lax.dot_general(
            ohk, hk, (((0,), (0,)), ((), ())),
            preferred_element_type=jnp.float32)
        cnt_b = cnt_b + jnp.sum(ohk, axis=0)[:, None]
    mean_pool = ssum / jnp.maximum(cnt_b, 1.0)
    rows = []
    for gidx in range(_G):
        masked = jnp.where(bf == gidx, h2f, -jnp.inf)
        m128 = jnp.max(masked, axis=0)
        r16 = m128[0:16]
        for k in range(1, 8):
            r16 = jnp.maximum(r16, m128[k * 16:(k + 1) * 16])
        rows.append(r16)
    max_pool = jnp.stack(rows, axis=0)
    pooled = jnp.concatenate([mean_pool, max_pool], axis=1)
    z1 = jax.nn.relu(jnp.dot(pooled, wp1_ref[...],
                             preferred_element_type=jnp.float32) + bp1_ref[...])
    z2 = jax.nn.relu(jnp.dot(z1, wp2_ref[...],
                             preferred_element_type=jnp.float32) + bp2_ref[...])
    out_ref[...] = z2


def _sc_gather(table, idx3):
    mesh = plsc.VectorSubcoreMesh(core_axis_name="c", subcore_axis_name="s")

    @functools.partial(
        pl.kernel,
        out_type=jax.ShapeDtypeStruct((_E, 16), jnp.float32),
        mesh=mesh,
        compiler_params=pltpu.CompilerParams(use_tc_tiling_on_sc=False),
        scratch_types=[
            pltpu.VMEM((_NCHUNK, _CH), jnp.int32),
            pltpu.VMEM((_EPT, 16), jnp.float32),
            pltpu.SemaphoreType.DMA,
        ],
    )
    def k(table_hbm, idx_hbm, out_hbm, idx_v, rows_v, sem):
        w = lax.axis_index("s") * _NC + lax.axis_index("c")
        pltpu.sync_copy(idx_hbm.at[w], idx_v)

        def body(j, _):
            pltpu.async_copy(table_hbm.at[idx_v.at[j]],
                             rows_v.at[pl.ds(j * _CH, _CH)], sem).wait()
            return ()

        lax.fori_loop(0, _NCHUNK, body, ())
        pltpu.sync_copy(rows_v, out_hbm.at[pl.ds(w * _EPT, _EPT)])

    return k(table, idx3)


def _sc_scatter(msg, idx3, zeros_blk, ones_blk, with_cnt):
    mesh = plsc.VectorSubcoreMesh(core_axis_name="c", subcore_axis_name="s")
    n_out = 2 if with_cnt else 1
    out_type = [jax.ShapeDtypeStruct((_NC, _N, 16), jnp.float32)] * n_out
    scratch = [
        pltpu.VMEM((_NCHUNK, _CH), jnp.int32),
        pltpu.VMEM((_EPT, 16), jnp.float32),
        pltpu.VMEM((_CH, 16), jnp.float32),
        pltpu.VMEM_SHARED((_N, 16), jnp.float32),
        pltpu.VMEM_SHARED((_N, 16), jnp.float32),
    ]

    @functools.partial(pl.kernel, out_type=out_type, mesh=mesh,
                       compiler_params=pltpu.CompilerParams(
                           use_tc_tiling_on_sc=False),
                       scratch_types=scratch)
    def k(msg_hbm, idx_hbm, zeros_hbm, ones_hbm, *refs):
        outs = refs[:n_out]
        idx_v, rows_v, ones_v, acc, cacc = refs[n_out:]
        cidx = lax.axis_index("c")
        sid = lax.axis_index("s")
        w = sid * _NC + cidx
        pltpu.sync_copy(zeros_hbm, acc.at[pl.ds(sid * _NPT, _NPT)])
        if with_cnt:
            pltpu.sync_copy(zeros_hbm, cacc.at[pl.ds(sid * _NPT, _NPT)])
        pltpu.sync_copy(idx_hbm.at[w], idx_v)
        pltpu.sync_copy(msg_hbm.at[pl.ds(w * _EPT, _EPT)], rows_v)
        if with_cnt:
            pltpu.sync_copy(ones_hbm, ones_v)
        plsc.subcore_barrier()

        def body(j, _):
            pltpu.sync_copy(rows_v.at[pl.ds(j * _CH, _CH)],
                            acc.at[idx_v.at[j]], add=True)
            if with_cnt:
                pltpu.sync_copy(ones_v, cacc.at[idx_v.at[j]], add=True)
            return ()

        lax.fori_loop(0, _NCHUNK, body, ())
        plsc.subcore_barrier()
        pltpu.sync_copy(acc.at[pl.ds(sid * _NPT, _NPT)],
                        outs[0].at[cidx].at[pl.ds(sid * _NPT, _NPT)])
        if with_cnt:
            pltpu.sync_copy(cacc.at[pl.ds(sid * _NPT, _NPT)],
                            outs[1].at[cidx].at[pl.ds(sid * _NPT, _NPT)])

    return k(msg, idx3, zeros_blk, ones_blk)


def _node_embed(x, w, b, g, be):
    return pl.pallas_call(
        _node_embed_body,
        out_shape=jax.ShapeDtypeStruct((_N, _H), jnp.float32),
    )(x, w, b, g, be)


def _edge_embed(eaf, w, b, g, be):
    nb = 10
    rows = _E8 // nb
    return pl.pallas_call(
        _edge_embed_body,
        grid=(nb,),
        in_specs=[
            pl.BlockSpec((rows, 128), lambda i: (i, 0)),
            pl.BlockSpec((_DE, _DE), lambda i: (0, 0)),
            pl.BlockSpec((_DE,), lambda i: (0,)),
            pl.BlockSpec((_DE,), lambda i: (0,)),
            pl.BlockSpec((_DE,), lambda i: (0,)),
        ],
        out_specs=[
            pl.BlockSpec((rows, 128), lambda i: (i, 0)),
            pl.BlockSpec((2, 16), lambda i: (0, 0)),
        ],
        out_shape=[
            jax.ShapeDtypeStruct((_E8, 128), jnp.float32),
            jax.ShapeDtypeStruct((2, 16), jnp.float32),
        ],
        scratch_shapes=[pltpu.VMEM((2, 128), jnp.float32)],
    )(eaf, w, b, g, be)


def _msg(zf, xjf, ec, wn, bn, act_tanh):
    nb = 10
    rows = _E8 // nb
    return pl.pallas_call(
        functools.partial(_msg_body, act_tanh),
        grid=(nb,),
        in_specs=[
            pl.BlockSpec((rows, 128), lambda i: (i, 0)),
            pl.BlockSpec((rows, 128), lambda i: (i, 0)),
            pl.BlockSpec((2, 16), lambda i: (0, 0)),
            pl.BlockSpec((16, 256), lambda i: (0, 0)),
            pl.BlockSpec((256,), lambda i: (0,)),
        ],
        out_specs=pl.BlockSpec((rows, 128), lambda i: (i, 0)),
        out_shape=jax.ShapeDtypeStruct((_E8, 128), jnp.float32),
    )(zf, xjf, ec, wn, bn)


def _update(h, sp, cp, root, bias, g, bb):
    return pl.pallas_call(
        _update_body,
        out_shape=jax.ShapeDtypeStruct((_N, _H), jnp.float32),
    )(h, sp, cp, root, bias, g, bb)


def _final(hf, spf, cpf, bf, root, bias, g, bb, wp1, bp1, wp2, bp2):
    return pl.pallas_call(
        _final_body,
        out_shape=jax.ShapeDtypeStruct((_G, 1), jnp.float32),
    )(hf, spf, cpf, bf, root, bias, g, bb, wp1, bp1, wp2, bp2)


def kernel(x, edge_index, edge_attr, batch, W_node, b_node, g_node, be_node,
           W_edge, b_edge, g_edge, be_edge, Wn1, bn1, root1, bias1, g1, bb1,
           Wn2, bn2, root2, bias2, g2, bb2, Wp1, bp1, Wp2, bp2):
    src3 = jnp.reshape(edge_index[0], (_NW, _NCHUNK, _CH))
    dst3 = jnp.reshape(edge_index[1], (_NW, _NCHUNK, _CH))
    eaf = jnp.reshape(edge_attr, (_E8, 128))
    zeros_blk = jnp.zeros((_NPT, 16), jnp.float32)
    ones_blk = jnp.ones((_CH, 16), jnp.float32)
    batch_f = jnp.reshape(jnp.repeat(batch, _H), (_N // 8, 128))

    h0 = _node_embed(x, W_node, b_node, g_node, be_node)
    zf, ec = _edge_embed(eaf, W_edge, b_edge, g_edge, be_edge)

    xj1 = _sc_gather(h0, src3)
    xjf1 = jnp.reshape(xj1, (_E8, 128))
    msg1 = jnp.reshape(_msg(zf, xjf1, ec, Wn1, bn1, True), (_E, 16))
    s1p, cp = _sc_scatter(msg1, dst3, zeros_blk, ones_blk, True)
    h1 = _update(h0, s1p, cp, root1, bias1, g1, bb1)

    xj2 = _sc_gather(h1, src3)
    xjf2 = jnp.reshape(xj2, (_E8, 128))
    msg2 = jnp.reshape(_msg(zf, xjf2, ec, Wn2, bn2, False), (_E, 16))
    (s2p,) = _sc_scatter(msg2, dst3, zeros_blk, ones_blk, False)

    h1f = jnp.reshape(h1, (_N // 8, 128))
    s2pf = jnp.reshape(s2p, (2, _N // 8, 128))
    cpf = jnp.reshape(cp, (2, _N // 8, 128))
    return _final(h1f, s2pf, cpf, batch_f, root2, bias2, g2, bb2,
                  Wp1, bp1, Wp2, bp2)


# speedup vs baseline: 7.3101x; 1.0738x over previous
import functools

import jax
import jax.numpy as jnp
from jax import lax
from jax.experimental import pallas as pl
from jax.experimental.pallas import tpu as pltpu
from jax.experimental.pallas import tpu_sc as plsc

_N = 10000
_E = 160000
_DF = 128
_DE = 16
_H = 16
_G = 64
_EPS = 1e-5

_NC = 2
_NS = 16
_NW = _NC * _NS
_EPT = _E // _NW
_CH = 125
_NCHUNK = _EPT // _CH
_NPT = _N // _NS

_E8 = _E // 8


def _node_embed_body(x_ref, w_ref, b_ref, g_ref, be_ref, out_ref):
    z = jax.nn.relu(jnp.dot(x_ref[...], w_ref[...],
                            preferred_element_type=jnp.float32) + b_ref[...])
    m = jnp.mean(z, axis=0, keepdims=True)
    v = jnp.mean(z * z, axis=0, keepdims=True) - m * m
    out_ref[...] = g_ref[...] * (z - m) * jax.lax.rsqrt(v + _EPS) + be_ref[...]


def _edge_embed_body(eaf_ref, w_ref, b_ref, g_ref, be_ref, zf_ref, ec_ref,
                     acc_ref):
    i = pl.program_id(0)
    nb = pl.num_programs(0)
    w16 = w_ref[...]
    wrow = jnp.concatenate([w16] * 8, axis=0)
    wbd_t = jnp.concatenate([w16.T] * 8, axis=0)
    wtile = jnp.concatenate([jnp.concatenate([w16] * 8, axis=1)] * 8, axis=0)
    rg = lax.broadcasted_iota(jnp.int32, (128, 128), 0) // 16
    cg = lax.broadcasted_iota(jnp.int32, (128, 128), 1) // 16
    wbd = jnp.where(rg == cg, wtile, 0.0)
    b128 = jnp.concatenate([b_ref[...]] * 8, axis=0)
    z = jax.nn.relu(jnp.dot(eaf_ref[...], wbd,
                            preferred_element_type=jnp.float32) + b128)
    zf_ref[...] = z

    @pl.when(i == 0)
    def _init():
        acc_ref[...] = jnp.zeros_like(acc_ref)

    s1 = jnp.sum(z, axis=0)
    s2 = jnp.sum(z * z, axis=0)
    acc_ref[...] = acc_ref[...] + jnp.stack([s1, s2], axis=0)

    @pl.when(i == nb - 1)
    def _fin():
        tot = acc_ref[...]
        m16 = jnp.zeros((16,), jnp.float32)
        q16 = jnp.zeros((16,), jnp.float32)
        for gidx in range(8):
            m16 = m16 + tot[0, gidx * 16:(gidx + 1) * 16]
            q16 = q16 + tot[1, gidx * 16:(gidx + 1) * 16]
        m = m16 / _E
        v = q16 / _E - m * m
        a = g_ref[...] * jax.lax.rsqrt(v + _EPS)
        c = be_ref[...] - m * a
        ec_ref[...] = jnp.stack([a, c], axis=0)


def _msg_body(act_tanh, zf_ref, xjf_ref, ec_ref, wn_ref, bn_ref, out_ref):
    a = ec_ref[0, :]
    c = ec_ref[1, :]
    wn_eff = a[:, None] * wn_ref[...]
    bn_eff = bn_ref[...] + jnp.dot(c[None, :], wn_ref[...],
                                   preferred_element_type=jnp.float32)[0]
    ri = lax.broadcasted_iota(jnp.int32, (16, 256), 0)
    rj = lax.broadcasted_iota(jnp.int32, (16, 256), 1)
    rmat = (rj // 16 == ri).astype(jnp.float32)
    si = lax.broadcasted_iota(jnp.int32, (256, 16), 0)
    sj = lax.broadcasted_iota(jnp.int32, (256, 16), 1)
    smat = (si % 16 == sj).astype(jnp.float32)
    slots = []
    for gidx in range(8):
        eg = zf_ref[:, gidx * 16:(gidx + 1) * 16]
        xg = xjf_ref[:, gidx * 16:(gidx + 1) * 16]
        t = jnp.dot(eg, wn_eff, preferred_element_type=jnp.float32) + bn_eff
        t = jnp.tanh(t) if act_tanh else jax.nn.relu(t)
        xr = jnp.dot(xg, rmat, preferred_element_type=jnp.float32)
        slots.append(jnp.dot(t * xr, smat, preferred_element_type=jnp.float32))
    out_ref[...] = jnp.concatenate(slots, axis=1)


def _update_body(h_ref, sp_ref, cp_ref, root_ref, bias_ref, g_ref, bb_ref,
                 out_ref):
    s = sp_ref[0] + sp_ref[1]
    cnt = cp_ref[0, :, 0:1] + cp_ref[1, :, 0:1]
    agg = s / jnp.maximum(cnt, 1.0)
    u = jnp.dot(h_ref[...], root_ref[...],
                preferred_element_type=jnp.float32) + agg + bias_ref[...]
    m = jnp.mean(u, axis=0, keepdims=True)
    v = jnp.mean(u * u, axis=0, keepdims=True) - m * m
    out_ref[...] = g_ref[...] * (u - m) * jax.lax.rsqrt(v + _EPS) + bb_ref[...]


def _final_body(hf_ref, spf_ref, cpf_ref, bf_ref, root_ref, bias_ref, g_ref,
                bb_ref, wp1_ref, bp1_ref, wp2_ref, bp2_ref, out_ref):
    root = root_ref[...]
    rtile = jnp.concatenate([jnp.concatenate([root] * 8, axis=1)] * 8, axis=0)
    rg = lax.broadcasted_iota(jnp.int32, (128, 128), 0) // 16
    cg = lax.broadcasted_iota(jnp.int32, (128, 128), 1) // 16
    rbd = jnp.where(rg == cg, rtile, 0.0)
    sf = spf_ref[0] + spf_ref[1]
    cntf = cpf_ref[0] + cpf_ref[1]
    aggf = sf / jnp.maximum(cntf, 1.0)
    bias128 = jnp.concatenate([bias_ref[...]] * 8, axis=0)
    uf = jnp.dot(hf_ref[...], rbd,
                 preferred_element_type=jnp.float32) + aggf + bias128
    s128 = jnp.sum(uf, axis=0)
    q128 = jnp.sum(uf * uf, axis=0)
    m16 = jnp.zeros((16,), jnp.float32)
    q16 = jnp.zeros((16,), jnp.float32)
    for k in range(8):
        m16 = m16 + s128[k * 16:(k + 1) * 16]
        q16 = q16 + q128[k * 16:(k + 1) * 16]
    m16 = m16 / _N
    v16 = q16 / _N - m16 * m16
    a16 = g_ref[...] * jax.lax.rsqrt(v16 + _EPS)
    c16 = bb_ref[...] - m16 * a16
    a128 = jnp.concatenate([a16] * 8, axis=0)
    c128 = jnp.concatenate([c16] * 8, axis=0)
    h2f = uf * a128 + c128

    bf = bf_ref[...]
    nr = _N // 8
    gids = lax.broadcasted_iota(jnp.int32, (nr, _G), 1)
    ssum = jnp.zeros((_G, _H), jnp.float32)
    cnt_b = jnp.zeros((_G, 1), jnp.float32)
    for k in range(8):
        bk = bf[:, k * 16:k * 16 + 1]
        ohk = (bk == gids).astype(jnp.float32)
        hk = h2f[:, k * 16:(k + 1) * 16]
        ssum = ssum + jax.lax.dot_general(
            ohk, hk, (((0,), (0,)), ((), ())),
            preferred_element_type=jnp.float32)
        cnt_b = cnt_b + jnp.sum(ohk, axis=0)[:, None]
    mean_pool = ssum / jnp.maximum(cnt_b, 1.0)
    rows = []
    for gidx in range(_G):
        masked = jnp.where(bf == gidx, h2f, -jnp.inf)
        m128 = jnp.max(masked, axis=0)
        r16 = m128[0:16]
        for k in range(1, 8):
            r16 = jnp.maximum(r16, m128[k * 16:(k + 1) * 16])
        rows.append(r16)
    max_pool = jnp.stack(rows, axis=0)
    pooled = jnp.concatenate([mean_pool, max_pool], axis=1)
    z1 = jax.nn.relu(jnp.dot(pooled, wp1_ref[...],
                             preferred_element_type=jnp.float32) + bp1_ref[...])
    z2 = jax.nn.relu(jnp.dot(z1, wp2_ref[...],
                             preferred_element_type=jnp.float32) + bp2_ref[...])
    out_ref[...] = z2


def _sc_gather(table, idx3):
    mesh = plsc.VectorSubcoreMesh(core_axis_name="c", subcore_axis_name="s")

    @functools.partial(
        pl.kernel,
        out_type=jax.ShapeDtypeStruct((_E, 16), jnp.float32),
        mesh=mesh,
        compiler_params=pltpu.CompilerParams(use_tc_tiling_on_sc=False),
        scratch_types=[
            pltpu.VMEM((_NCHUNK, _CH), jnp.int32),
            pltpu.VMEM((_EPT, 16), jnp.float32),
            pltpu.SemaphoreType.DMA,
        ],
    )
    def k(table_hbm, idx_hbm, out_hbm, idx_v, rows_v, sem):
        w = lax.axis_index("s") * _NC + lax.axis_index("c")
        pltpu.sync_copy(idx_hbm.at[w], idx_v)

        def body(j, _):
            pltpu.async_copy(table_hbm.at[idx_v.at[j]],
                             rows_v.at[pl.ds(j * _CH, _CH)], sem)
            return ()

        lax.fori_loop(0, _NCHUNK, body, ())
        pltpu.make_async_copy(out_hbm.at[pl.ds(w * _EPT, _EPT)], rows_v,
                              sem).wait()
        pltpu.sync_copy(rows_v, out_hbm.at[pl.ds(w * _EPT, _EPT)])

    return k(table, idx3)


def _sc_scatter(msg, idx3, zeros_blk, ones_blk, with_cnt):
    mesh = plsc.VectorSubcoreMesh(core_axis_name="c", subcore_axis_name="s")
    n_out = 2 if with_cnt else 1
    out_type = [jax.ShapeDtypeStruct((_NC, _N, 16), jnp.float32)] * n_out
    scratch = [
        pltpu.VMEM((_NCHUNK, _CH), jnp.int32),
        pltpu.VMEM((_EPT, 16), jnp.float32),
        pltpu.VMEM((_CH, 16), jnp.float32),
        pltpu.VMEM_SHARED((_N, 16), jnp.float32),
        pltpu.VMEM_SHARED((_N, 16), jnp.float32),
        pltpu.SemaphoreType.DMA,
        pltpu.SemaphoreType.DMA,
    ]

    @functools.partial(pl.kernel, out_type=out_type, mesh=mesh,
                       compiler_params=pltpu.CompilerParams(
                           use_tc_tiling_on_sc=False),
                       scratch_types=scratch)
    def k(msg_hbm, idx_hbm, zeros_hbm, ones_hbm, *refs):
        outs = refs[:n_out]
        idx_v, rows_v, ones_v, acc, cacc, sem, sem2 = refs[n_out:]
        cidx = lax.axis_index("c")
        sid = lax.axis_index("s")
        w = sid * _NC + cidx
        pltpu.sync_copy(zeros_hbm, acc.at[pl.ds(sid * _NPT, _NPT)])
        if with_cnt:
            pltpu.sync_copy(zeros_hbm, cacc.at[pl.ds(sid * _NPT, _NPT)])
        pltpu.sync_copy(idx_hbm.at[w], idx_v)
        pltpu.sync_copy(msg_hbm.at[pl.ds(w * _EPT, _EPT)], rows_v)
        if with_cnt:
            pltpu.sync_copy(ones_hbm, ones_v)
        plsc.subcore_barrier()

        def body(j, _):
            pltpu.async_copy(rows_v.at[pl.ds(j * _CH, _CH)],
                             acc.at[idx_v.at[j]], sem, add=True)
            if with_cnt:
                pltpu.async_copy(ones_v, cacc.at[idx_v.at[j]], sem2, add=True)
            return ()

        lax.fori_loop(0, _NCHUNK, body, ())
        pltpu.make_async_copy(msg_hbm.at[pl.ds(w * _EPT, _EPT)], rows_v,
                              sem).wait()
        if with_cnt:
            pltpu.make_async_copy(msg_hbm.at[pl.ds(w * _EPT, _EPT)], rows_v,
                                  sem2).wait()
        plsc.subcore_barrier()
        pltpu.sync_copy(acc.at[pl.ds(sid * _NPT, _NPT)],
                        outs[0].at[cidx].at[pl.ds(sid * _NPT, _NPT)])
        if with_cnt:
            pltpu.sync_copy(cacc.at[pl.ds(sid * _NPT, _NPT)],
                            outs[1].at[cidx].at[pl.ds(sid * _NPT, _NPT)])

    return k(msg, idx3, zeros_blk, ones_blk)


def _node_embed(x, w, b, g, be):
    return pl.pallas_call(
        _node_embed_body,
        out_shape=jax.ShapeDtypeStruct((_N, _H), jnp.float32),
    )(x, w, b, g, be)


def _edge_embed(eaf, w, b, g, be):
    nb = 10
    rows = _E8 // nb
    return pl.pallas_call(
        _edge_embed_body,
        grid=(nb,),
        in_specs=[
            pl.BlockSpec((rows, 128), lambda i: (i, 0)),
            pl.BlockSpec((_DE, _DE), lambda i: (0, 0)),
            pl.BlockSpec((_DE,), lambda i: (0,)),
            pl.BlockSpec((_DE,), lambda i: (0,)),
            pl.BlockSpec((_DE,), lambda i: (0,)),
        ],
        out_specs=[
            pl.BlockSpec((rows, 128), lambda i: (i, 0)),
            pl.BlockSpec((2, 16), lambda i: (0, 0)),
        ],
        out_shape=[
            jax.ShapeDtypeStruct((_E8, 128), jnp.float32),
            jax.ShapeDtypeStruct((2, 16), jnp.float32),
        ],
        scratch_shapes=[pltpu.VMEM((2, 128), jnp.float32)],
    )(eaf, w, b, g, be)


def _msg(zf, xjf, ec, wn, bn, act_tanh):
    nb = 10
    rows = _E8 // nb
    return pl.pallas_call(
        functools.partial(_msg_body, act_tanh),
        grid=(nb,),
        in_specs=[
            pl.BlockSpec((rows, 128), lambda i: (i, 0)),
            pl.BlockSpec((rows, 128), lambda i: (i, 0)),
            pl.BlockSpec((2, 16), lambda i: (0, 0)),
            pl.BlockSpec((16, 256), lambda i: (0, 0)),
            pl.BlockSpec((256,), lambda i: (0,)),
        ],
        out_specs=pl.BlockSpec((rows, 128), lambda i: (i, 0)),
        out_shape=jax.ShapeDtypeStruct((_E8, 128), jnp.float32),
    )(zf, xjf, ec, wn, bn)


def _update(h, sp, cp, root, bias, g, bb):
    return pl.pallas_call(
        _update_body,
        out_shape=jax.ShapeDtypeStruct((_N, _H), jnp.float32),
    )(h, sp, cp, root, bias, g, bb)


def _final(hf, spf, cpf, bf, root, bias, g, bb, wp1, bp1, wp2, bp2):
    return pl.pallas_call(
        _final_body,
        out_shape=jax.ShapeDtypeStruct((_G, 1), jnp.float32),
    )(hf, spf, cpf, bf, root, bias, g, bb, wp1, bp1, wp2, bp2)


def kernel(x, edge_index, edge_attr, batch, W_node, b_node, g_node, be_node,
           W_edge, b_edge, g_edge, be_edge, Wn1, bn1, root1, bias1, g1, bb1,
           Wn2, bn2, root2, bias2, g2, bb2, Wp1, bp1, Wp2, bp2):
    src3 = jnp.reshape(edge_index[0], (_NW, _NCHUNK, _CH))
    dst3 = jnp.reshape(edge_index[1], (_NW, _NCHUNK, _CH))
    eaf = jnp.reshape(edge_attr, (_E8, 128))
    zeros_blk = jnp.zeros((_NPT, 16), jnp.float32)
    ones_blk = jnp.ones((_CH, 16), jnp.float32)
    batch_f = jnp.reshape(jnp.repeat(batch, _H), (_N // 8, 128))

    h0 = _node_embed(x, W_node, b_node, g_node, be_node)
    zf, ec = _edge_embed(eaf, W_edge, b_edge, g_edge, be_edge)

    xj1 = _sc_gather(h0, src3)
    xjf1 = jnp.reshape(xj1, (_E8, 128))
    msg1 = jnp.reshape(_msg(zf, xjf1, ec, Wn1, bn1, True), (_E, 16))
    s1p, cp = _sc_scatter(msg1, dst3, zeros_blk, ones_blk, True)
    h1 = _update(h0, s1p, cp, root1, bias1, g1, bb1)

    xj2 = _sc_gather(h1, src3)
    xjf2 = jnp.reshape(xj2, (_E8, 128))
    msg2 = jnp.reshape(_msg(zf, xjf2, ec, Wn2, bn2, False), (_E, 16))
    (s2p,) = _sc_scatter(msg2, dst3, zeros_blk, ones_blk, False)

    h1f = jnp.reshape(h1, (_N // 8, 128))
    s2pf = jnp.reshape(s2p, (2, _N // 8, 128))
    cpf = jnp.reshape(cp, (2, _N // 8, 128))
    return _final(h1f, s2pf, cpf, batch_f, root2, bias2, g2, bb2,
                  Wp1, bp1, Wp2, bp2)
